# baseline (speedup 1.0000x reference)
import jax
import jax.numpy as jnp
from jax.experimental import pallas as pl
from jax.experimental.pallas import tpu as pltpu

_GRID = 15
_BU = 66688
_BI = 6784


def _copy_body(u_in, i_in, u_out, i_out):
    u_out[...] = u_in[...]
    i_out[...] = i_in[...]


def kernel(emb_user, emb_item):
    ut = emb_user.T
    it = emb_item.T
    u, i = pl.pallas_call(
        _copy_body,
        grid=(_GRID,),
        in_specs=[
            pl.BlockSpec((32, _BU), lambda g: (0, g)),
            pl.BlockSpec((32, _BI), lambda g: (0, g)),
        ],
        out_specs=[
            pl.BlockSpec((32, _BU), lambda g: (0, g)),
            pl.BlockSpec((32, _BI), lambda g: (0, g)),
        ],
        out_shape=[
            jax.ShapeDtypeStruct(ut.shape, ut.dtype),
            jax.ShapeDtypeStruct(it.shape, it.dtype),
        ],
    )(ut, it)
    return (u.T, i.T)


# speedup vs baseline: 1.0030x; 1.0030x over previous
import jax
import jax.numpy as jnp
from jax.experimental import pallas as pl
from jax.experimental.pallas import tpu as pltpu

_GRID = 10
_BU = 100096
_BI = 10112


def _copy_body(u_in, i_in, u_out, i_out):
    u_out[...] = u_in[...]
    i_out[...] = i_in[...]


def kernel(emb_user, emb_item):
    ut = emb_user.T
    it = emb_item.T
    u, i = pl.pallas_call(
        _copy_body,
        grid=(_GRID,),
        in_specs=[
            pl.BlockSpec((32, _BU), lambda g: (0, g)),
            pl.BlockSpec((32, _BI), lambda g: (0, g)),
        ],
        out_specs=[
            pl.BlockSpec((32, _BU), lambda g: (0, g)),
            pl.BlockSpec((32, _BI), lambda g: (0, g)),
        ],
        out_shape=[
            jax.ShapeDtypeStruct(ut.shape, ut.dtype),
            jax.ShapeDtypeStruct(it.shape, it.dtype),
        ],
    )(ut, it)
    return (u.T, i.T)


# speedup vs baseline: 1.0048x; 1.0017x over previous
import jax
from jax.experimental import pallas as pl

_GRID = 16
_BU = 65536
_BI = 6656


def _copy_body(u_in, i_in, u_out, i_out):
    u_out[...] = u_in[...]
    i_out[...] = i_in[...]


def kernel(emb_user, emb_item):
    ut = emb_user.T
    it = emb_item.T
    u, i = pl.pallas_call(
        _copy_body,
        grid=(_GRID,),
        in_specs=[
            pl.BlockSpec((32, _BU), lambda g: (0, g)),
            pl.BlockSpec((32, _BI), lambda g: (0, g)),
        ],
        out_specs=[
            pl.BlockSpec((32, _BU), lambda g: (0, g)),
            pl.BlockSpec((32, _BI), lambda g: (0, g)),
        ],
        out_shape=[
            jax.ShapeDtypeStruct(ut.shape, ut.dtype),
            jax.ShapeDtypeStruct(it.shape, it.dtype),
        ],
    )(ut, it)
    return (u.T, i.T)
